# baseline (device time: 490701 ns/iter reference)
import jax
import jax.numpy as jnp
from jax import lax
from jax.experimental import pallas as pl
from jax.experimental.pallas import tpu as pltpu

N_DEV = 32
N_TOK = 1024
D_MODEL = 256
N_EXPERTS = 128
D_OUT = 512
E_LOCAL = N_EXPERTS // N_DEV
CHUNK = N_TOK // N_DEV


def kernel(x, router_W, route_idx, expert_W):
    def body(x_ref, rw_ref, idx_ref, ew_ref, out_ref,
             comm_ref, send_sems, recv_sems, credit_sem):
        p = lax.axis_index("i")
        left = jnp.mod(p - 1, N_DEV)
        right = jnp.mod(p + 1, N_DEV)

        xf = x_ref[...]
        scores = jnp.dot(xf, rw_ref[...], preferred_element_type=jnp.float32)
        s_max = jnp.max(scores, axis=-1, keepdims=True)
        e = jnp.exp(scores - s_max)
        probs = e / jnp.sum(e, axis=-1, keepdims=True)

        cols = lax.broadcasted_iota(jnp.int32, (N_TOK, N_EXPERTS), 1)
        top2 = jnp.logical_or(cols == idx_ref[:, 0:1], cols == idx_ref[:, 1:2])
        sel = jnp.where(top2, probs, 0.0)
        w = sel / jnp.sum(sel, axis=-1, keepdims=True)

        partial = jnp.zeros((N_TOK, D_OUT), jnp.float32)
        for j in range(E_LOCAL):
            eid = p * E_LOCAL + j
            gate = jnp.sum(jnp.where(cols == eid, w, 0.0), axis=-1, keepdims=True)
            xg = (xf * gate).astype(jnp.bfloat16)
            wj = ew_ref[j].astype(jnp.bfloat16)
            partial = partial + jnp.dot(xg, wj, preferred_element_type=jnp.float32)
        out_ref[...] = partial

        barrier_sem = pltpu.get_barrier_semaphore()
        pl.semaphore_signal(barrier_sem, inc=1, device_id=(left,),
                            device_id_type=pl.DeviceIdType.MESH)
        pl.semaphore_signal(barrier_sem, inc=1, device_id=(right,),
                            device_id_type=pl.DeviceIdType.MESH)
        pl.semaphore_wait(barrier_sem, 2)

        for h in range(2 * (N_DEV - 1)):
            slot = h % 2
            send_row = jnp.mod(p - h, N_DEV) * CHUNK
            recv_row = jnp.mod(p - h - 1, N_DEV) * CHUNK
            rdma = pltpu.make_async_remote_copy(
                src_ref=out_ref.at[pl.ds(send_row, CHUNK), :],
                dst_ref=comm_ref.at[slot],
                send_sem=send_sems.at[slot],
                recv_sem=recv_sems.at[slot],
                device_id=(right,),
                device_id_type=pl.DeviceIdType.MESH,
            )
            rdma.start()
            rdma.wait()
            if h < N_DEV - 1:
                out_ref[pl.ds(recv_row, CHUNK), :] += comm_ref[slot]
            else:
                out_ref[pl.ds(recv_row, CHUNK), :] = comm_ref[slot]
            pl.semaphore_signal(credit_sem, inc=1, device_id=(left,),
                                device_id_type=pl.DeviceIdType.MESH)
            pl.semaphore_wait(credit_sem, 1)

    return pl.pallas_call(
        body,
        out_shape=jax.ShapeDtypeStruct((N_TOK, D_OUT), jnp.float32),
        in_specs=[pl.BlockSpec(memory_space=pltpu.VMEM)] * 4,
        out_specs=pl.BlockSpec(memory_space=pltpu.VMEM),
        scratch_shapes=[
            pltpu.VMEM((2, CHUNK, D_OUT), jnp.float32),
            pltpu.SemaphoreType.DMA((2,)),
            pltpu.SemaphoreType.DMA((2,)),
            pltpu.SemaphoreType.REGULAR,
        ],
        compiler_params=pltpu.CompilerParams(collective_id=0),
    )(x, router_W, route_idx, expert_W)


# device time: 77531 ns/iter; 6.3291x vs baseline; 6.3291x over previous
import jax
import jax.numpy as jnp
from jax import lax
from jax.experimental import pallas as pl
from jax.experimental.pallas import tpu as pltpu

N_DEV = 32
N_TOK = 1024
D_MODEL = 256
N_EXPERTS = 128
D_OUT = 512
E_LOCAL = N_EXPERTS // N_DEV

BITS = (8, 1, 2, 4, 16)
RS_ROWS = (512, 256, 128, 64, 32)
RS_OFF = (0, 512, 768, 896, 960)
AG_ROWS = (32, 64, 128, 256, 512)
AG_OFF = (0, 32, 96, 224, 480)


def kernel(x, router_W, route_idx, expert_W):
    def body(x_ref, rw_ref, idx_ref, ew_ref, out_ref,
             rs_comm, ag_comm, send_sems, recv_sems):
        p = lax.axis_index("i")

        xf = x_ref[...]
        scores = jnp.dot(xf, rw_ref[...], preferred_element_type=jnp.float32)
        s_max = jnp.max(scores, axis=-1, keepdims=True)
        e = jnp.exp(scores - s_max)
        probs = e / jnp.sum(e, axis=-1, keepdims=True)

        cols = lax.broadcasted_iota(jnp.int32, (N_TOK, N_EXPERTS), 1)
        top2 = jnp.logical_or(cols == idx_ref[:, 0:1], cols == idx_ref[:, 1:2])
        sel = jnp.where(top2, probs, 0.0)
        w = sel / jnp.sum(sel, axis=-1, keepdims=True)

        partial = jnp.zeros((N_TOK, D_OUT), jnp.float32)
        for j in range(E_LOCAL):
            eid = p * E_LOCAL + j
            gate = jnp.sum(jnp.where(cols == eid, w, 0.0), axis=-1, keepdims=True)
            xg = (xf * gate).astype(jnp.bfloat16)
            wj = ew_ref[j].astype(jnp.bfloat16)
            partial = partial + jnp.dot(xg, wj, preferred_element_type=jnp.float32)
        out_ref[...] = partial

        barrier_sem = pltpu.get_barrier_semaphore()
        for b in BITS:
            pl.semaphore_signal(barrier_sem, inc=1,
                                device_id=(jnp.bitwise_xor(p, b),),
                                device_id_type=pl.DeviceIdType.MESH)
        pl.semaphore_wait(barrier_sem, len(BITS))

        start = jnp.int32(0)
        for k, b in enumerate(BITS):
            half = RS_ROWS[k]
            partner = jnp.bitwise_xor(p, b)
            upper = (jnp.bitwise_and(p, b) != 0).astype(jnp.int32)
            send_off = pl.multiple_of(start + (1 - upper) * half, 32)
            keep_off = pl.multiple_of(start + upper * half, 32)
            rdma = pltpu.make_async_remote_copy(
                src_ref=out_ref.at[pl.ds(send_off, half), :],
                dst_ref=rs_comm.at[pl.ds(RS_OFF[k], half), :],
                send_sem=send_sems.at[k],
                recv_sem=recv_sems.at[k],
                device_id=(partner,),
                device_id_type=pl.DeviceIdType.MESH,
            )
            rdma.start()
            rdma.wait()
            out_ref[pl.ds(keep_off, half), :] += rs_comm[
                pl.ds(RS_OFF[k], half), :]
            start = keep_off

        for j in range(len(BITS)):
            b = BITS[len(BITS) - 1 - j]
            size = AG_ROWS[j]
            partner = jnp.bitwise_xor(p, b)
            upper = (jnp.bitwise_and(p, b) != 0).astype(jnp.int32)
            sibling = pl.multiple_of(start + (1 - 2 * upper) * size, 32)
            start = pl.multiple_of(start, 32)
            rdma = pltpu.make_async_remote_copy(
                src_ref=out_ref.at[pl.ds(start, size), :],
                dst_ref=ag_comm.at[pl.ds(AG_OFF[j], size), :],
                send_sem=send_sems.at[len(BITS) + j],
                recv_sem=recv_sems.at[len(BITS) + j],
                device_id=(partner,),
                device_id_type=pl.DeviceIdType.MESH,
            )
            rdma.start()
            rdma.wait()
            out_ref[pl.ds(sibling, size), :] = ag_comm[
                pl.ds(AG_OFF[j], size), :]
            start = jnp.minimum(start, sibling)

    n_sems = 2 * len(BITS)
    return pl.pallas_call(
        body,
        out_shape=jax.ShapeDtypeStruct((N_TOK, D_OUT), jnp.float32),
        in_specs=[pl.BlockSpec(memory_space=pltpu.VMEM)] * 4,
        out_specs=pl.BlockSpec(memory_space=pltpu.VMEM),
        scratch_shapes=[
            pltpu.VMEM((1024, D_OUT), jnp.float32),
            pltpu.VMEM((1024, D_OUT), jnp.float32),
            pltpu.SemaphoreType.DMA((n_sems,)),
            pltpu.SemaphoreType.DMA((n_sems,)),
        ],
        compiler_params=pltpu.CompilerParams(collective_id=0),
    )(x, router_W, route_idx, expert_W)


# device time: 54253 ns/iter; 9.0447x vs baseline; 1.4291x over previous
import jax
import jax.numpy as jnp
from jax import lax
from jax.experimental import pallas as pl
from jax.experimental.pallas import tpu as pltpu

N_DEV = 32
N_TOK = 1024
D_MODEL = 256
N_EXPERTS = 128
D_OUT = 512
E_LOCAL = N_EXPERTS // N_DEV

BITS = (8, 1, 2, 4, 16)
RS_ROWS = (512, 256, 128, 64, 32)
RS_OFF = (0, 512, 768, 896, 960)
AG_ROWS = (32, 64, 128, 256, 512)
AG_OFF = (0, 32, 96, 224, 480)


def kernel(x, router_W, route_idx, expert_W):
    def body(x_ref, rw_ref, idx_ref, ew_ref, out_ref,
             acc_ref, rs_comm, ag_comm, send_sems, recv_sems):
        p = lax.axis_index("i")

        xf = x_ref[...]
        scores = jnp.dot(xf, rw_ref[...], preferred_element_type=jnp.float32)
        s_max = jnp.max(scores, axis=-1, keepdims=True)
        e = jnp.exp(scores - s_max)
        probs = e / jnp.sum(e, axis=-1, keepdims=True)

        cols = lax.broadcasted_iota(jnp.int32, (N_TOK, N_EXPERTS), 1)
        top2 = jnp.logical_or(cols == idx_ref[:, 0:1], cols == idx_ref[:, 1:2])
        sel = jnp.where(top2, probs, 0.0)
        w = sel / jnp.sum(sel, axis=-1, keepdims=True)

        partial = jnp.zeros((N_TOK, D_OUT), jnp.float32)
        for j in range(E_LOCAL):
            eid = p * E_LOCAL + j
            gate = jnp.sum(jnp.where(cols == eid, w, 0.0), axis=-1, keepdims=True)
            xg = (xf * gate).astype(jnp.bfloat16)
            wj = ew_ref[j].astype(jnp.bfloat16)
            partial = partial + jnp.dot(xg, wj, preferred_element_type=jnp.float32)
        acc_ref[...] = partial.astype(jnp.bfloat16)

        barrier_sem = pltpu.get_barrier_semaphore()
        for b in BITS:
            pl.semaphore_signal(barrier_sem, inc=1,
                                device_id=(jnp.bitwise_xor(p, b),),
                                device_id_type=pl.DeviceIdType.MESH)
        pl.semaphore_wait(barrier_sem, len(BITS))

        start = jnp.int32(0)
        for k, b in enumerate(BITS):
            half = RS_ROWS[k]
            partner = jnp.bitwise_xor(p, b)
            upper = (jnp.bitwise_and(p, b) != 0).astype(jnp.int32)
            send_off = pl.multiple_of(start + (1 - upper) * half, 32)
            keep_off = pl.multiple_of(start + upper * half, 32)
            rdma = pltpu.make_async_remote_copy(
                src_ref=acc_ref.at[pl.ds(send_off, half), :],
                dst_ref=rs_comm.at[pl.ds(RS_OFF[k], half), :],
                send_sem=send_sems.at[k],
                recv_sem=recv_sems.at[k],
                device_id=(partner,),
                device_id_type=pl.DeviceIdType.MESH,
            )
            rdma.start()
            rdma.wait()
            acc_ref[pl.ds(keep_off, half), :] += rs_comm[
                pl.ds(RS_OFF[k], half), :]
            start = keep_off

        for j in range(len(BITS)):
            b = BITS[len(BITS) - 1 - j]
            size = AG_ROWS[j]
            partner = jnp.bitwise_xor(p, b)
            upper = (jnp.bitwise_and(p, b) != 0).astype(jnp.int32)
            sibling = pl.multiple_of(start + (1 - 2 * upper) * size, 32)
            start = pl.multiple_of(start, 32)
            rdma = pltpu.make_async_remote_copy(
                src_ref=acc_ref.at[pl.ds(start, size), :],
                dst_ref=ag_comm.at[pl.ds(AG_OFF[j], size), :],
                send_sem=send_sems.at[len(BITS) + j],
                recv_sem=recv_sems.at[len(BITS) + j],
                device_id=(partner,),
                device_id_type=pl.DeviceIdType.MESH,
            )
            rdma.start()
            rdma.wait()
            acc_ref[pl.ds(sibling, size), :] = ag_comm[
                pl.ds(AG_OFF[j], size), :]
            start = jnp.minimum(start, sibling)

        out_ref[...] = acc_ref[...].astype(jnp.float32)

    n_sems = 2 * len(BITS)
    return pl.pallas_call(
        body,
        out_shape=jax.ShapeDtypeStruct((N_TOK, D_OUT), jnp.float32),
        in_specs=[pl.BlockSpec(memory_space=pltpu.VMEM)] * 4,
        out_specs=pl.BlockSpec(memory_space=pltpu.VMEM),
        scratch_shapes=[
            pltpu.VMEM((N_TOK, D_OUT), jnp.bfloat16),
            pltpu.VMEM((1024, D_OUT), jnp.bfloat16),
            pltpu.VMEM((1024, D_OUT), jnp.bfloat16),
            pltpu.SemaphoreType.DMA((n_sems,)),
            pltpu.SemaphoreType.DMA((n_sems,)),
        ],
        compiler_params=pltpu.CompilerParams(collective_id=0),
    )(x, router_W, route_idx, expert_W)


# device time: 54047 ns/iter; 9.0792x vs baseline; 1.0038x over previous
import jax
import jax.numpy as jnp
from jax import lax
from jax.experimental import pallas as pl
from jax.experimental.pallas import tpu as pltpu

N_DEV = 32
N_TOK = 1024
D_MODEL = 256
N_EXPERTS = 128
D_OUT = 512
E_LOCAL = N_EXPERTS // N_DEV

BITS = (8, 1, 2, 4, 16)
RS_ROWS = (512, 256, 128, 64, 32)
RS_OFF = (0, 512, 768, 896, 960)
AG_ROWS = (32, 64, 128, 256, 512)
AG_OFF = (0, 32, 96, 224, 480)


def kernel(x, router_W, route_idx, expert_W):
    def body(x_ref, rw_ref, idx_ref, ew_ref, out_ref,
             acc_ref, w_ref, send_buf, rs_comm, ag_comm, send_sems, recv_sems):
        p = lax.axis_index("i")

        xf = x_ref[...]
        scores = jnp.dot(xf, rw_ref[...], preferred_element_type=jnp.float32)
        s_max = jnp.max(scores, axis=-1, keepdims=True)
        e = jnp.exp(scores - s_max)
        probs = e / jnp.sum(e, axis=-1, keepdims=True)

        cols = lax.broadcasted_iota(jnp.int32, (N_TOK, N_EXPERTS), 1)
        top2 = jnp.logical_or(cols == idx_ref[:, 0:1], cols == idx_ref[:, 1:2])
        sel = jnp.where(top2, probs, 0.0)
        w_ref[...] = sel / jnp.sum(sel, axis=-1, keepdims=True)

        def partial_for(off, nrows):
            xs = x_ref[pl.ds(off, nrows), :]
            ws = w_ref[pl.ds(off, nrows), :]
            ecols = lax.broadcasted_iota(jnp.int32, (nrows, N_EXPERTS), 1)
            acc = jnp.zeros((nrows, D_OUT), jnp.float32)
            for j in range(E_LOCAL):
                eid = p * E_LOCAL + j
                gate = jnp.sum(jnp.where(ecols == eid, ws, 0.0),
                               axis=-1, keepdims=True)
                xg = (xs * gate).astype(jnp.bfloat16)
                wj = ew_ref[j].astype(jnp.bfloat16)
                acc = acc + jnp.dot(xg, wj, preferred_element_type=jnp.float32)
            return acc.astype(jnp.bfloat16)

        b0 = BITS[0]
        half0 = RS_ROWS[0]
        upper0 = (jnp.bitwise_and(p, b0) != 0).astype(jnp.int32)
        send_off0 = pl.multiple_of((1 - upper0) * half0, 32)
        keep_off0 = pl.multiple_of(upper0 * half0, 32)
        send_buf[...] = partial_for(send_off0, half0)

        barrier_sem = pltpu.get_barrier_semaphore()
        for b in BITS:
            pl.semaphore_signal(barrier_sem, inc=1,
                                device_id=(jnp.bitwise_xor(p, b),),
                                device_id_type=pl.DeviceIdType.MESH)
        pl.semaphore_wait(barrier_sem, len(BITS))

        rdma0 = pltpu.make_async_remote_copy(
            src_ref=send_buf,
            dst_ref=rs_comm.at[pl.ds(RS_OFF[0], half0), :],
            send_sem=send_sems.at[0],
            recv_sem=recv_sems.at[0],
            device_id=(jnp.bitwise_xor(p, b0),),
            device_id_type=pl.DeviceIdType.MESH,
        )
        rdma0.start()
        acc_ref[pl.ds(keep_off0, half0), :] = partial_for(keep_off0, half0)
        rdma0.wait()
        acc_ref[pl.ds(keep_off0, half0), :] += rs_comm[
            pl.ds(RS_OFF[0], half0), :]
        start = keep_off0

        for k in range(1, len(BITS)):
            b = BITS[k]
            half = RS_ROWS[k]
            partner = jnp.bitwise_xor(p, b)
            upper = (jnp.bitwise_and(p, b) != 0).astype(jnp.int32)
            send_off = pl.multiple_of(start + (1 - upper) * half, 32)
            keep_off = pl.multiple_of(start + upper * half, 32)
            rdma = pltpu.make_async_remote_copy(
                src_ref=acc_ref.at[pl.ds(send_off, half), :],
                dst_ref=rs_comm.at[pl.ds(RS_OFF[k], half), :],
                send_sem=send_sems.at[k],
                recv_sem=recv_sems.at[k],
                device_id=(partner,),
                device_id_type=pl.DeviceIdType.MESH,
            )
            rdma.start()
            rdma.wait()
            acc_ref[pl.ds(keep_off, half), :] += rs_comm[
                pl.ds(RS_OFF[k], half), :]
            start = keep_off

        for j in range(len(BITS)):
            b = BITS[len(BITS) - 1 - j]
            size = AG_ROWS[j]
            partner = jnp.bitwise_xor(p, b)
            upper = (jnp.bitwise_and(p, b) != 0).astype(jnp.int32)
            sibling = pl.multiple_of(start + (1 - 2 * upper) * size, 32)
            start = pl.multiple_of(start, 32)
            rdma = pltpu.make_async_remote_copy(
                src_ref=acc_ref.at[pl.ds(start, size), :],
                dst_ref=ag_comm.at[pl.ds(AG_OFF[j], size), :],
                send_sem=send_sems.at[len(BITS) + j],
                recv_sem=recv_sems.at[len(BITS) + j],
                device_id=(partner,),
                device_id_type=pl.DeviceIdType.MESH,
            )
            rdma.start()
            rdma.wait()
            acc_ref[pl.ds(sibling, size), :] = ag_comm[
                pl.ds(AG_OFF[j], size), :]
            start = jnp.minimum(start, sibling)

        out_ref[...] = acc_ref[...].astype(jnp.float32)

    n_sems = 2 * len(BITS)
    return pl.pallas_call(
        body,
        out_shape=jax.ShapeDtypeStruct((N_TOK, D_OUT), jnp.float32),
        in_specs=[pl.BlockSpec(memory_space=pltpu.VMEM)] * 4,
        out_specs=pl.BlockSpec(memory_space=pltpu.VMEM),
        scratch_shapes=[
            pltpu.VMEM((N_TOK, D_OUT), jnp.bfloat16),
            pltpu.VMEM((N_TOK, N_EXPERTS), jnp.float32),
            pltpu.VMEM((RS_ROWS[0], D_OUT), jnp.bfloat16),
            pltpu.VMEM((1024, D_OUT), jnp.bfloat16),
            pltpu.VMEM((1024, D_OUT), jnp.bfloat16),
            pltpu.SemaphoreType.DMA((n_sems,)),
            pltpu.SemaphoreType.DMA((n_sems,)),
        ],
        compiler_params=pltpu.CompilerParams(collective_id=0),
    )(x, router_W, route_idx, expert_W)
